# baseline (speedup 1.0000x reference)
import functools

import jax
import jax.numpy as jnp
from jax import lax
from jax.experimental import pallas as pl
from jax.experimental.pallas import tpu as pltpu
from jax.experimental.pallas import tpu_sc as plsc

T, A, D = 32, 32, 2048
E, H, NA, ADD = 8, 2048, 32, 12
N = T * A
TM = 128
G = 15
CAP = 2048

NC, NS = 2, 16
NW = NC * NS
OUT_W = 128
KB = 10
BD = (D + ADD) // KB


def _sc_gather_rows(table, idx, rows_per_worker, chunk):
    B = idx.shape[0]
    row_shape = table.shape[1:]
    nch = rows_per_worker // chunk
    mesh = plsc.VectorSubcoreMesh(core_axis_name="c", subcore_axis_name="s")

    @functools.partial(
        pl.kernel,
        mesh=mesh,
        out_type=jax.ShapeDtypeStruct((B,) + row_shape, table.dtype),
        scratch_types=[
            pltpu.VMEM((rows_per_worker,), jnp.int32),
            pltpu.VMEM((chunk,) + row_shape, table.dtype),
            pltpu.VMEM((chunk,) + row_shape, table.dtype),
            pltpu.SemaphoreType.DMA,
            pltpu.SemaphoreType.DMA,
            pltpu.SemaphoreType.DMA,
            pltpu.SemaphoreType.DMA,
        ],
    )
    def k(table_hbm, idx_hbm, out_hbm, idx_v, buf_a, buf_b, gs_a, gs_b,
          ss_a, ss_b):
        wid = lax.axis_index("s") * NC + lax.axis_index("c")
        base = wid * rows_per_worker
        pltpu.sync_copy(idx_hbm.at[pl.ds(base, rows_per_worker)], idx_v)
        bufs, gsems, ssems = [buf_a, buf_b], [gs_a, gs_b], [ss_a, ss_b]
        gathers = [None] * nch
        stores = [None] * nch
        gathers[0] = pltpu.async_copy(
            table_hbm.at[idx_v.at[pl.ds(0, chunk)]], bufs[0], gsems[0])
        for c in range(nch):
            b = c % 2
            gathers[c].wait()
            if c + 1 < nch:
                if c >= 1:
                    stores[c - 1].wait()
                gathers[c + 1] = pltpu.async_copy(
                    table_hbm.at[idx_v.at[pl.ds((c + 1) * chunk, chunk)]],
                    bufs[(c + 1) % 2], gsems[(c + 1) % 2])
            stores[c] = pltpu.async_copy(
                bufs[b], out_hbm.at[pl.ds(base + c * chunk, chunk)], ssems[b])
        stores[nch - 1].wait()
        if nch >= 2:
            stores[nch - 2].wait()

    return k(table, idx)


def _tc_expert_tiles(te, valid, pos, xkb, W1x, W2, b1, b2, ph2f):
    def body(te_ref, valid_ref, pos_ref, x_ref, w1_ref, w2_ref, b1_ref,
             b2_ref, ph2f_ref, out_ref, hacc_ref):
        kb = pl.program_id(0)
        w = pl.program_id(1)
        e = te_ref[w]
        hrows = pl.ds(w * TM, TM)

        @pl.when(valid_ref[w] == 1)
        def _():
            row_ids = jax.lax.broadcasted_iota(jnp.int32, (TM, N), 0) + w * TM
            mask = (row_ids == pos_ref[0][None, :]).astype(jnp.bfloat16)
            x_t = jnp.dot(mask, x_ref[0],
                          preferred_element_type=jnp.float32)
            x_t = x_t.astype(jnp.bfloat16)
            w1slab = w1_ref[:, e, :].astype(jnp.bfloat16)
            contrib = jnp.dot(x_t, w1slab,
                              preferred_element_type=jnp.float32)

            @pl.when(kb == 0)
            def _():
                hacc_ref[hrows, :] = contrib

            @pl.when(kb > 0)
            def _():
                hacc_ref[hrows, :] = hacc_ref[hrows, :] + contrib

            @pl.when(kb == KB - 1)
            def _():
                b1eff = b1_ref[0, 0]
                for a in range(ADD):
                    b1eff = b1eff + ph2f_ref[e, a] * w1_ref[BD - ADD + a, e, :]
                h = jnp.maximum(hacc_ref[hrows, :] + b1eff[None, :], 0.0)
                logits = jnp.dot(h.astype(jnp.bfloat16), w2_ref[0],
                                 preferred_element_type=jnp.float32)
                logits = logits + b2_ref[0, 0][None, :]
                m = jnp.max(logits, axis=1, keepdims=True)
                lse = jnp.log(jnp.sum(jnp.exp(logits - m), axis=1,
                                      keepdims=True))
                out_ref[:, NA:] = jnp.zeros((TM, OUT_W - NA), jnp.float32)
                out_ref[:, :NA] = logits - (m + lse)

        @pl.when(valid_ref[w] == 0)
        def _():
            out_ref[...] = jnp.zeros_like(out_ref)

    grid_spec = pltpu.PrefetchScalarGridSpec(
        num_scalar_prefetch=2,
        grid=(KB, G),
        in_specs=[
            pl.BlockSpec((1, N), lambda kb, w, te, v: (0, 0)),
            pl.BlockSpec((1, N, BD), lambda kb, w, te, v: (kb, 0, 0)),
            pl.BlockSpec((BD, E, H), lambda kb, w, te, v: (kb, 0, 0)),
            pl.BlockSpec((1, H, NA), lambda kb, w, te, v: (te[w], 0, 0)),
            pl.BlockSpec((1, 1, H), lambda kb, w, te, v: (te[w], 0, 0)),
            pl.BlockSpec((1, 1, NA), lambda kb, w, te, v: (te[w], 0, 0)),
            pl.BlockSpec(memory_space=pltpu.SMEM),
        ],
        out_specs=pl.BlockSpec((TM, OUT_W), lambda kb, w, te, v: (w, 0)),
        scratch_shapes=[pltpu.VMEM((G * TM, H), jnp.float32)],
    )
    return pl.pallas_call(
        body,
        grid_spec=grid_spec,
        out_shape=jax.ShapeDtypeStruct((G * TM, OUT_W), jnp.float32),
        compiler_params=pltpu.CompilerParams(
            dimension_semantics=("arbitrary", "arbitrary"),
        ),
    )(te, valid, pos, xkb, W1x, W2, b1.reshape(E, 1, H),
      b2.reshape(E, 1, NA), ph2f)


def kernel(obs, expert_ids, ph_to_feature, W1, b1, W2, b2):
    x_bf = obs.reshape(N, D).astype(jnp.bfloat16)
    eid = expert_ids.reshape(-1).astype(jnp.int32)

    onehot = (eid[:, None] == jnp.arange(E, dtype=jnp.int32)[None, :])
    onehot = onehot.astype(jnp.int32)
    counts = jnp.sum(onehot, axis=0)
    ranks_all = jnp.cumsum(onehot, axis=0) - onehot
    rank = jnp.sum(ranks_all * onehot, axis=1)
    tiles_per_e = (counts + TM - 1) // TM
    ctiles = jnp.cumsum(tiles_per_e)
    tile_start_e = ctiles - tiles_per_e
    pos = jnp.sum(onehot * tile_start_e[None, :], axis=1) * TM + rank
    total_tiles = ctiles[E - 1]
    t_arr = jnp.arange(G, dtype=jnp.int32)
    te_raw = jnp.sum((ctiles[None, :] <= t_arr[:, None]).astype(jnp.int32),
                     axis=1)
    valid = (t_arr < total_tiles).astype(jnp.int32)
    last_e = jnp.sum((ctiles <= total_tiles - 1).astype(jnp.int32))
    te = jnp.where(valid == 1, jnp.minimum(te_raw, E - 1), last_e)

    W1x = jnp.transpose(W1, (1, 0, 2))
    xp = jnp.pad(x_bf, ((0, 0), (0, KB * BD - D)))
    xkb = jnp.transpose(xp.reshape(N, KB, BD), (1, 0, 2))
    out_sorted = _tc_expert_tiles(te, valid, pos.reshape(1, N), xkb,
                                  W1x, W2.astype(jnp.bfloat16),
                                  b1, b2, ph_to_feature)

    logp = _sc_gather_rows(out_sorted, pos.astype(jnp.int32),
                           rows_per_worker=N // NW, chunk=N // NW)
    return logp[:, :NA].reshape(T, A, NA)


# speedup vs baseline: 2.6123x; 2.6123x over previous
import functools

import jax
import jax.numpy as jnp
from jax import lax
from jax.experimental import pallas as pl
from jax.experimental.pallas import tpu as pltpu
from jax.experimental.pallas import tpu_sc as plsc

T, A, D = 32, 32, 2048
E, H, NA, ADD = 8, 2048, 32, 12
N = T * A
TM = 128
G = 15
CAP = 2048

NC, NS = 2, 16
NW = NC * NS
OUT_W = 128
TMLOG = 7


def _sc_gather_rows(table, idx, rows_per_worker, chunk):
    B = idx.shape[0]
    row_shape = table.shape[1:]
    nch = rows_per_worker // chunk
    mesh = plsc.VectorSubcoreMesh(core_axis_name="c", subcore_axis_name="s")

    @functools.partial(
        pl.kernel,
        mesh=mesh,
        out_type=jax.ShapeDtypeStruct((B,) + row_shape, table.dtype),
        scratch_types=[
            pltpu.VMEM((rows_per_worker,), jnp.int32),
            pltpu.VMEM((chunk,) + row_shape, table.dtype),
            pltpu.VMEM((chunk,) + row_shape, table.dtype),
            pltpu.SemaphoreType.DMA,
            pltpu.SemaphoreType.DMA,
            pltpu.SemaphoreType.DMA,
            pltpu.SemaphoreType.DMA,
        ],
    )
    def k(table_hbm, idx_hbm, out_hbm, idx_v, buf_a, buf_b, gs_a, gs_b,
          ss_a, ss_b):
        wid = lax.axis_index("s") * NC + lax.axis_index("c")
        base = wid * rows_per_worker
        pltpu.sync_copy(idx_hbm.at[pl.ds(base, rows_per_worker)], idx_v)
        bufs, gsems, ssems = [buf_a, buf_b], [gs_a, gs_b], [ss_a, ss_b]
        gathers = [None] * nch
        stores = [None] * nch
        gathers[0] = pltpu.async_copy(
            table_hbm.at[idx_v.at[pl.ds(0, chunk)]], bufs[0], gsems[0])
        for c in range(nch):
            b = c % 2
            gathers[c].wait()
            if c + 1 < nch:
                if c >= 1:
                    stores[c - 1].wait()
                gathers[c + 1] = pltpu.async_copy(
                    table_hbm.at[idx_v.at[pl.ds((c + 1) * chunk, chunk)]],
                    bufs[(c + 1) % 2], gsems[(c + 1) % 2])
            stores[c] = pltpu.async_copy(
                bufs[b], out_hbm.at[pl.ds(base + c * chunk, chunk)], ssems[b])
        stores[nch - 1].wait()
        if nch >= 2:
            stores[nch - 2].wait()

    return k(table, idx)


def _tc_routing_meta(eid2d, x2d):
    def body(eid_ref, x_ref, pos_ref, te_ref, valid_ref, xbf_ref):
        eid = eid_ref[0]
        e_iota = jax.lax.broadcasted_iota(jnp.int32, (E, N), 0)
        onehot = (eid[None, :] == e_iota).astype(jnp.int32)
        cum = onehot
        s = 1
        while s < N:
            shifted = jnp.concatenate(
                [jnp.zeros((E, s), jnp.int32), cum[:, :N - s]], axis=1)
            cum = cum + shifted
            s *= 2
        counts = cum[:, N - 1:N]
        tiles = jax.lax.shift_right_logical(counts + (TM - 1), TMLOG)
        ct = tiles
        s = 1
        while s < E:
            ct = ct + jnp.concatenate(
                [jnp.zeros((s, 1), jnp.int32), ct[:E - s, :]], axis=0)
            s *= 2
        tstart = ct - tiles
        rank = jnp.sum((cum - onehot) * onehot, axis=0)
        pos_ref[0, :] = jnp.sum(onehot * tstart, axis=0) * TM + rank
        t_arr = jax.lax.broadcasted_iota(jnp.int32, (1, G), 1)
        te_raw = jnp.sum((ct <= t_arr).astype(jnp.int32), axis=0,
                         keepdims=True)
        total = ct[E - 1:E, :]
        valid = (t_arr < total).astype(jnp.int32)
        last_e = jnp.sum((ct <= total - 1).astype(jnp.int32), axis=0,
                         keepdims=True)
        te_ref[...] = jnp.where(valid == 1, jnp.minimum(te_raw, E - 1),
                                jnp.broadcast_to(last_e, (1, G)))
        valid_ref[...] = valid
        xbf_ref[...] = x_ref[...].astype(jnp.bfloat16)

    return pl.pallas_call(
        body,
        out_shape=[
            jax.ShapeDtypeStruct((1, N), jnp.int32),
            jax.ShapeDtypeStruct((1, G), jnp.int32),
            jax.ShapeDtypeStruct((1, G), jnp.int32),
            jax.ShapeDtypeStruct((N, D), jnp.bfloat16),
        ],
    )(eid2d, x2d)


def _tc_expert_tiles(te, valid, pos, x_bf, W1, W2, b1, b2, ph2f):
    def body(te_ref, valid_ref, pos_ref, x_ref, w1_ref, w2_ref, b1_ref,
             b2_ref, ph2f_ref, out_ref):
        w = pl.program_id(0)
        e = te_ref[w]

        @pl.when(valid_ref[w] == 1)
        def _():
            row_ids = jax.lax.broadcasted_iota(jnp.int32, (TM, N), 0) + w * TM
            mask = (row_ids == pos_ref[0][None, :]).astype(jnp.bfloat16)
            x_tile = jnp.dot(mask, x_ref[...],
                             preferred_element_type=jnp.float32)
            x_tile = x_tile.astype(jnp.bfloat16)
            b1eff = b1_ref[0, 0]
            for a in range(ADD):
                b1eff = b1eff + ph2f_ref[e, a] * w1_ref[0, D + a, :]
            h = jnp.dot(x_tile, w1_ref[0, :D, :],
                        preferred_element_type=jnp.float32)
            h = jnp.maximum(h + b1eff[None, :], 0.0)
            logits = jnp.dot(h.astype(jnp.bfloat16), w2_ref[0],
                             preferred_element_type=jnp.float32)
            logits = logits + b2_ref[0, 0][None, :]
            m = jnp.max(logits, axis=1, keepdims=True)
            lse = jnp.log(jnp.sum(jnp.exp(logits - m), axis=1, keepdims=True))
            out_ref[:, NA:] = jnp.zeros((TM, OUT_W - NA), jnp.float32)
            out_ref[:, :NA] = logits - (m + lse)

        @pl.when(valid_ref[w] == 0)
        def _():
            out_ref[...] = jnp.zeros_like(out_ref)

    grid_spec = pltpu.PrefetchScalarGridSpec(
        num_scalar_prefetch=2,
        grid=(G,),
        in_specs=[
            pl.BlockSpec((1, N), lambda w, te, v: (0, 0)),
            pl.BlockSpec((N, D), lambda w, te, v: (0, 0)),
            pl.BlockSpec((1, D + ADD, H), lambda w, te, v: (te[w], 0, 0)),
            pl.BlockSpec((1, H, NA), lambda w, te, v: (te[w], 0, 0)),
            pl.BlockSpec((1, 1, H), lambda w, te, v: (te[w], 0, 0)),
            pl.BlockSpec((1, 1, NA), lambda w, te, v: (te[w], 0, 0)),
            pl.BlockSpec(memory_space=pltpu.SMEM),
        ],
        out_specs=pl.BlockSpec((TM, OUT_W), lambda w, te, v: (w, 0)),
    )
    return pl.pallas_call(
        body,
        grid_spec=grid_spec,
        out_shape=jax.ShapeDtypeStruct((G * TM, OUT_W), jnp.float32),
        compiler_params=pltpu.CompilerParams(
            dimension_semantics=("arbitrary",),
        ),
    )(te, valid, pos, x_bf, W1, W2, b1.reshape(E, 1, H),
      b2.reshape(E, 1, NA), ph2f)


def kernel(obs, expert_ids, ph_to_feature, W1, b1, W2, b2):
    x2d = obs.reshape(N, D)
    eid2d = expert_ids.reshape(1, N).astype(jnp.int32)

    pos2d, te2d, valid2d, x_bf = _tc_routing_meta(eid2d, x2d)
    te = te2d.reshape(G)
    valid = valid2d.reshape(G)

    out_sorted = _tc_expert_tiles(te, valid, pos2d, x_bf,
                                  W1.astype(jnp.bfloat16),
                                  W2.astype(jnp.bfloat16),
                                  b1, b2, ph_to_feature)

    logp = _sc_gather_rows(out_sorted, pos2d.reshape(N),
                           rows_per_worker=N // NW, chunk=N // NW)
    return logp[:, :NA].reshape(T, A, NA)
